# baseline (speedup 1.0000x reference)
import functools

import jax
import jax.numpy as jnp
from jax import lax
from jax.experimental import pallas as pl
from jax.experimental.pallas import tpu as pltpu
from jax.experimental.pallas import tpu_sc as plsc

USER_N = 15000
ITEM_N = 35000
N_NODES = USER_N + ITEM_N
D = 64
E = 800000

HALF = N_NODES // 2
PAD = 88
HALF_P = HALF + PAD
N_P = 2 * HALF_P

NSUB = 16
RPT = HALF_P // NSUB
WCH = 49
NWCH = RPT // WCH

ECH = 128
EPT = 50176
E_P = NSUB * EPT
NCH = EPT // ECH
NPAIR = NCH // 2

_mesh = plsc.VectorSubcoreMesh(core_axis_name="c", subcore_axis_name="s")

_DNUMS = lax.GatherDimensionNumbers(
    offset_dims=(), collapsed_slice_dims=(0,), start_index_map=(0,))


def _adjust(ebf, eidx, ev, vrow):
    for g in range(ECH // 16):
        sl = pl.ds(g * 16, 16)
        eidx[0, sl] = ebf[0, sl].astype(jnp.int32)
        eidx[1, sl] = ebf[1, sl].astype(jnp.int32)
        ev[sl] = ebf[vrow, sl]


def _scale(buf, ev):
    for g in range(ECH // 16):
        v16 = ev[pl.ds(g * 16, 16)]
        for l in range(16):
            vb = lax.gather(
                v16, jnp.full((16, 1), l, jnp.int32), _DNUMS, (1,),
                mode=lax.GatherScatterMode.PROMISE_IN_BOUNDS)
            r = g * 16 + l
            for c in range(4):
                sl = pl.ds(c * 16, 16)
                buf[r, sl] = buf[r, sl] * vb


def _layer_body(do_mean, ego, edata, e1, e2, out,
                acc, ebf0, ebf1, eidx0, eidx1, ev0, ev1, g0, g1, ba, bb, bc,
                si0, si1, sg0, sg1, ss0, ss1):
    cid = lax.axis_index("c")
    sid = lax.axis_index("s")
    vrow = 2 + cid

    cbase = sid * NCH
    pltpu.async_copy(edata.at[cbase], ebf0, si0)
    pltpu.async_copy(edata.at[cbase + 1], ebf1, si1)

    z16 = jnp.zeros((16,), jnp.float32)

    def zrow(r, carry):
        for c in range(4):
            ba[r, pl.ds(c * 16, 16)] = z16
        return carry

    lax.fori_loop(0, WCH, zrow, 0)
    zsem = (sg0, sg1, ss0, ss1)
    for j in range(NWCH):
        pltpu.async_copy(ba, acc.at[pl.ds(sid * RPT + j * WCH, WCH)],
                         zsem[j % 4])
    for j in range(NWCH):
        pltpu.make_async_copy(
            ba, acc.at[pl.ds(sid * RPT + j * WCH, WCH)], zsem[j % 4]).wait()
    plsc.subcore_barrier()


    def pair(i, carry):
        c0 = cbase + i * 2
        pltpu.make_async_copy(edata.at[c0], ebf0, si0).wait()

        @pl.when(i > 0)
        def _():
            pltpu.make_async_copy(g0, acc.at[eidx0.at[0]], ss0).wait()

        _adjust(ebf0, eidx0, ev0, vrow)
        gd0 = pltpu.async_copy(ego.at[eidx0.at[1]], g0, sg0)
        pltpu.async_copy(edata.at[c0 + 2], ebf0, si0)
        pltpu.make_async_copy(edata.at[c0 + 1], ebf1, si1).wait()

        @pl.when(i > 0)
        def _():
            pltpu.make_async_copy(g1, acc.at[eidx1.at[0]], ss1).wait()

        _adjust(ebf1, eidx1, ev1, vrow)
        gd1 = pltpu.async_copy(ego.at[eidx1.at[1]], g1, sg1)
        pltpu.async_copy(edata.at[c0 + 3], ebf1, si1)
        gd0.wait()
        _scale(g0, ev0)
        pltpu.async_copy(g0, acc.at[eidx0.at[0]], ss0, add=True)
        gd1.wait()
        _scale(g1, ev1)
        pltpu.async_copy(g1, acc.at[eidx1.at[0]], ss1, add=True)
        return carry

    lax.fori_loop(0, NPAIR, pair, 0)
    pltpu.make_async_copy(g0, acc.at[eidx0.at[0]], ss0).wait()
    pltpu.make_async_copy(g1, acc.at[eidx1.at[0]], ss1).wait()
    pltpu.make_async_copy(edata.at[cbase + NCH], ebf0, si0).wait()
    pltpu.make_async_copy(edata.at[cbase + NCH + 1], ebf1, si1).wait()
    plsc.subcore_barrier()

    obase = cid * HALF_P + sid * RPT
    if not do_mean:
        pltpu.sync_copy(acc.at[pl.ds(sid * RPT, RPT)],
                        out.at[pl.ds(obase, RPT)])
    else:
        inv3 = jnp.float32(1.0 / 3.0)
        for j in range(NWCH):
            a0 = sid * RPT + j * WCH
            o0 = obase + j * WCH
            da = pltpu.async_copy(acc.at[pl.ds(a0, WCH)], ba, si0)
            db = pltpu.async_copy(e1.at[pl.ds(o0, WCH)], bb, si1)
            dc = pltpu.async_copy(e2.at[pl.ds(o0, WCH)], bc, sg0)
            da.wait()
            db.wait()
            dc.wait()

            def mrow(r, carry):
                for c in range(4):
                    sl = pl.ds(c * 16, 16)
                    ba[r, sl] = (ba[r, sl] + bb[r, sl] + bc[r, sl]) * inv3
                return carry

            lax.fori_loop(0, WCH, mrow, 0)
            pltpu.sync_copy(ba, out.at[pl.ds(o0, WCH)])


def _make_layer(do_mean):
    return pl.kernel(
        functools.partial(_layer_body, do_mean),
        out_type=jax.ShapeDtypeStruct((N_P, D), jnp.float32),
        mesh=_mesh,
        compiler_params=pltpu.CompilerParams(use_tc_tiling_on_sc=False),
        scratch_types=[
            pltpu.VMEM_SHARED((HALF_P, D), jnp.float32),
            pltpu.VMEM((4, ECH), jnp.float32),
            pltpu.VMEM((4, ECH), jnp.float32),
            pltpu.VMEM((2, ECH), jnp.int32),
            pltpu.VMEM((2, ECH), jnp.int32),
            pltpu.VMEM((ECH,), jnp.float32),
            pltpu.VMEM((ECH,), jnp.float32),
            pltpu.VMEM((ECH, D), jnp.float32),
            pltpu.VMEM((ECH, D), jnp.float32),
            pltpu.VMEM((WCH, D), jnp.float32),
            pltpu.VMEM((WCH, D), jnp.float32),
            pltpu.VMEM((WCH, D), jnp.float32),
            pltpu.SemaphoreType.DMA,
            pltpu.SemaphoreType.DMA,
            pltpu.SemaphoreType.DMA,
            pltpu.SemaphoreType.DMA,
            pltpu.SemaphoreType.DMA,
            pltpu.SemaphoreType.DMA,
        ],
    )


_layer = _make_layer(False)
_layer_mean = _make_layer(True)


@jax.jit
def _run(user_emb, item_emb, adj_indices, adj_values):
    rows = adj_indices[0].astype(jnp.int32)
    cols = adj_indices[1].astype(jnp.int32)
    vals = adj_values.astype(jnp.float32)
    locs = (rows % HALF).astype(jnp.float32)
    colp = jnp.where(cols >= HALF, cols + PAD, cols).astype(jnp.float32)
    v0 = jnp.where(rows < HALF, vals, 0.0)
    v1 = jnp.where(rows >= HALF, vals, 0.0)
    epad = E_P + 2 * ECH - E
    z = jnp.zeros((epad,), jnp.float32)
    edata = jnp.stack(
        [jnp.concatenate([locs, z]).reshape(-1, ECH),
         jnp.concatenate([colp, z]).reshape(-1, ECH),
         jnp.concatenate([v0, z]).reshape(-1, ECH),
         jnp.concatenate([v1, z]).reshape(-1, ECH)], axis=1)
    zpad = jnp.zeros((PAD, D), jnp.float32)
    ego0 = jnp.concatenate(
        [user_emb, item_emb[:HALF - USER_N], zpad,
         item_emb[HALF - USER_N:], zpad], axis=0)
    e1 = _layer(ego0, edata, ego0, ego0)
    e2 = _layer(e1, edata, ego0, ego0)
    e3m = _layer_mean(e2, edata, e1, e2)
    user_all = e3m[:USER_N]
    item_all = jnp.concatenate(
        [e3m[USER_N:HALF], e3m[HALF_P:HALF_P + HALF]], axis=0)
    return user_all, item_all


def kernel(user_emb, item_emb, adj_indices, adj_values):
    return _run(user_emb, item_emb, adj_indices, adj_values)


# speedup vs baseline: 1.0328x; 1.0328x over previous
import functools

import jax
import jax.numpy as jnp
from jax import lax
from jax.experimental import pallas as pl
from jax.experimental.pallas import tpu as pltpu
from jax.experimental.pallas import tpu_sc as plsc

USER_N = 15000
ITEM_N = 35000
N_NODES = USER_N + ITEM_N
D = 64
E = 800000

HALF = N_NODES // 2
PAD = 88
HALF_P = HALF + PAD
N_P = 2 * HALF_P

NSUB = 16
RPT = HALF_P // NSUB
WCH = 49
NWCH = RPT // WCH

ECH = 128
EPT = 50176
E_P = NSUB * EPT
NCH = EPT // ECH
NPAIR = NCH // 2

_mesh = plsc.VectorSubcoreMesh(core_axis_name="c", subcore_axis_name="s")

_DNUMS = lax.GatherDimensionNumbers(
    offset_dims=(), collapsed_slice_dims=(0,), start_index_map=(0,))


def _adjust(ebf, eidx, ev, basef):
    halff = jnp.float32(HALF)
    for g in range(ECH // 16):
        sl = pl.ds(g * 16, 16)
        r16 = ebf[0, sl]
        loc = r16 - basef
        ok = (loc >= 0.0) & (loc < halff)
        loc = jnp.where(loc < 0.0, loc + halff, loc)
        loc = jnp.where(loc >= halff, loc - halff, loc)
        eidx[0, sl] = loc.astype(jnp.int32)
        c16 = ebf[1, sl]
        c16 = jnp.where(c16 >= halff, c16 + jnp.float32(PAD), c16)
        eidx[1, sl] = c16.astype(jnp.int32)
        v16 = ebf[2, sl]
        ev[sl] = jnp.where(ok, v16, jnp.float32(0.0))


def _scale(buf, ev):
    for g in range(ECH // 16):
        v16 = ev[pl.ds(g * 16, 16)]
        for l in range(16):
            vb = lax.gather(
                v16, jnp.full((16, 1), l, jnp.int32), _DNUMS, (1,),
                mode=lax.GatherScatterMode.PROMISE_IN_BOUNDS)
            r = g * 16 + l
            for c in range(4):
                sl = pl.ds(c * 16, 16)
                buf[r, sl] = buf[r, sl] * vb


def _layer_body(do_mean, ego, edata, e1, e2, out,
                acc, ebf0, ebf1, eidx0, eidx1, ev0, ev1, g0, g1, ba, bb, bc,
                si0, si1, sg0, sg1, ss0, ss1):
    cid = lax.axis_index("c")
    sid = lax.axis_index("s")
    basef = (cid * HALF).astype(jnp.float32)

    cbase = sid * NCH
    pltpu.async_copy(edata.at[cbase], ebf0, si0)
    pltpu.async_copy(edata.at[cbase + 1], ebf1, si1)

    z16 = jnp.zeros((16,), jnp.float32)

    def zrow(r, carry):
        for c in range(4):
            ba[r, pl.ds(c * 16, 16)] = z16
        return carry

    lax.fori_loop(0, WCH, zrow, 0)
    zsem = (sg0, sg1, ss0, ss1)
    for j in range(NWCH):
        pltpu.async_copy(ba, acc.at[pl.ds(sid * RPT + j * WCH, WCH)],
                         zsem[j % 4])
    for j in range(NWCH):
        pltpu.make_async_copy(
            ba, acc.at[pl.ds(sid * RPT + j * WCH, WCH)], zsem[j % 4]).wait()
    plsc.subcore_barrier()


    def pair(i, carry):
        c0 = cbase + i * 2
        pltpu.make_async_copy(edata.at[c0], ebf0, si0).wait()

        @pl.when(i > 0)
        def _():
            pltpu.make_async_copy(g0, acc.at[eidx0.at[0]], ss0).wait()

        _adjust(ebf0, eidx0, ev0, basef)
        gd0 = pltpu.async_copy(ego.at[eidx0.at[1]], g0, sg0)
        pltpu.async_copy(edata.at[c0 + 2], ebf0, si0)
        pltpu.make_async_copy(edata.at[c0 + 1], ebf1, si1).wait()

        @pl.when(i > 0)
        def _():
            pltpu.make_async_copy(g1, acc.at[eidx1.at[0]], ss1).wait()

        _adjust(ebf1, eidx1, ev1, basef)
        gd1 = pltpu.async_copy(ego.at[eidx1.at[1]], g1, sg1)
        pltpu.async_copy(edata.at[c0 + 3], ebf1, si1)
        gd0.wait()
        _scale(g0, ev0)
        pltpu.async_copy(g0, acc.at[eidx0.at[0]], ss0, add=True)
        gd1.wait()
        _scale(g1, ev1)
        pltpu.async_copy(g1, acc.at[eidx1.at[0]], ss1, add=True)
        return carry

    lax.fori_loop(0, NPAIR, pair, 0)
    pltpu.make_async_copy(g0, acc.at[eidx0.at[0]], ss0).wait()
    pltpu.make_async_copy(g1, acc.at[eidx1.at[0]], ss1).wait()
    pltpu.make_async_copy(edata.at[cbase + NCH], ebf0, si0).wait()
    pltpu.make_async_copy(edata.at[cbase + NCH + 1], ebf1, si1).wait()
    plsc.subcore_barrier()

    obase = cid * HALF_P + sid * RPT
    if not do_mean:
        pltpu.sync_copy(acc.at[pl.ds(sid * RPT, RPT)],
                        out.at[pl.ds(obase, RPT)])
    else:
        inv3 = jnp.float32(1.0 / 3.0)
        for j in range(NWCH):
            a0 = sid * RPT + j * WCH
            o0 = obase + j * WCH
            da = pltpu.async_copy(acc.at[pl.ds(a0, WCH)], ba, si0)
            db = pltpu.async_copy(e1.at[pl.ds(o0, WCH)], bb, si1)
            dc = pltpu.async_copy(e2.at[pl.ds(o0, WCH)], bc, sg0)
            da.wait()
            db.wait()
            dc.wait()

            def mrow(r, carry):
                for c in range(4):
                    sl = pl.ds(c * 16, 16)
                    ba[r, sl] = (ba[r, sl] + bb[r, sl] + bc[r, sl]) * inv3
                return carry

            lax.fori_loop(0, WCH, mrow, 0)
            pltpu.sync_copy(ba, out.at[pl.ds(o0, WCH)])


def _make_layer(do_mean):
    return pl.kernel(
        functools.partial(_layer_body, do_mean),
        out_type=jax.ShapeDtypeStruct((N_P, D), jnp.float32),
        mesh=_mesh,
        compiler_params=pltpu.CompilerParams(use_tc_tiling_on_sc=False),
        scratch_types=[
            pltpu.VMEM_SHARED((HALF_P, D), jnp.float32),
            pltpu.VMEM((3, ECH), jnp.float32),
            pltpu.VMEM((3, ECH), jnp.float32),
            pltpu.VMEM((2, ECH), jnp.int32),
            pltpu.VMEM((2, ECH), jnp.int32),
            pltpu.VMEM((ECH,), jnp.float32),
            pltpu.VMEM((ECH,), jnp.float32),
            pltpu.VMEM((ECH, D), jnp.float32),
            pltpu.VMEM((ECH, D), jnp.float32),
            pltpu.VMEM((WCH, D), jnp.float32),
            pltpu.VMEM((WCH, D), jnp.float32),
            pltpu.VMEM((WCH, D), jnp.float32),
            pltpu.SemaphoreType.DMA,
            pltpu.SemaphoreType.DMA,
            pltpu.SemaphoreType.DMA,
            pltpu.SemaphoreType.DMA,
            pltpu.SemaphoreType.DMA,
            pltpu.SemaphoreType.DMA,
        ],
    )


_layer = _make_layer(False)
_layer_mean = _make_layer(True)


@jax.jit
def _run(user_emb, item_emb, adj_indices, adj_values):
    rows = adj_indices[0].astype(jnp.float32)
    cols = adj_indices[1].astype(jnp.float32)
    vals = adj_values.astype(jnp.float32)
    epad = E_P + 2 * ECH - E
    rows_p = jnp.concatenate([rows, jnp.zeros((epad,), jnp.float32)])
    cols_p = jnp.concatenate([cols, jnp.zeros((epad,), jnp.float32)])
    vals_p = jnp.concatenate([vals, jnp.zeros((epad,), jnp.float32)])
    edata = jnp.stack(
        [rows_p.reshape(-1, ECH), cols_p.reshape(-1, ECH),
         vals_p.reshape(-1, ECH)], axis=1)
    zpad = jnp.zeros((PAD, D), jnp.float32)
    ego0 = jnp.concatenate(
        [user_emb, item_emb[:HALF - USER_N], zpad,
         item_emb[HALF - USER_N:], zpad], axis=0)
    e1 = _layer(ego0, edata, ego0, ego0)
    e2 = _layer(e1, edata, ego0, ego0)
    e3m = _layer_mean(e2, edata, e1, e2)
    user_all = e3m[:USER_N]
    item_all = jnp.concatenate(
        [e3m[USER_N:HALF], e3m[HALF_P:HALF_P + HALF]], axis=0)
    return user_all, item_all


def kernel(user_emb, item_emb, adj_indices, adj_values):
    return _run(user_emb, item_emb, adj_indices, adj_values)
